# baseline (speedup 1.0000x reference)
import jax
import jax.numpy as jnp
from jax import lax
from jax.experimental import pallas as pl
from jax.experimental.pallas import tpu as pltpu
from jax.experimental.pallas import tpu_sc as plsc

_T = 20
_N = 10000
_D = 128
_R = _T * _N

_BR = 25000
_GRID = _R // _BR

_NW = 32
_RC = 250
_NCHUNK = _R // _RC
_CPW = _NCHUNK // _NW
_CB = _RC * _D
_OC = 256


def _tc_body(obs_ref, nd_ref, spikes_ref, out_ref):
    obs = obs_ref[0]
    a = (obs == 1).astype(jnp.float32)
    b = (obs == -1).astype(jnp.float32)
    coef = jnp.concatenate([a, b], axis=0)
    contrib = jax.lax.dot_general(
        coef, spikes_ref[...],
        dimension_numbers=(((0,), (0,)), ((), ())),
        preferred_element_type=jnp.float32,
    )
    out_ref[...] = nd_ref[...] + contrib


def _tc_kernel(node_data, pos_test_spike, neg_test_spike, observations):
    nd = node_data.reshape(_R, _D)
    obs = observations.reshape(_GRID, 1, _BR).astype(jnp.int32)
    spikes = jnp.stack([pos_test_spike, neg_test_spike], axis=0)

    out = pl.pallas_call(
        _tc_body,
        grid=(_GRID,),
        in_specs=[
            pl.BlockSpec((1, 1, _BR), lambda i: (i, 0, 0)),
            pl.BlockSpec((_BR, _D), lambda i: (i, 0)),
            pl.BlockSpec((2, _D), lambda i: (0, 0)),
        ],
        out_specs=pl.BlockSpec((_BR, _D), lambda i: (i, 0)),
        out_shape=jax.ShapeDtypeStruct((_R, _D), jnp.float32),
        compiler_params=pltpu.CompilerParams(
            dimension_semantics=("arbitrary",),
        ),
    )(obs, nd, spikes)
    return out.reshape(_T, _N, _D)


def _sc_body(node_ref, obs_ref, pos_ref, neg_ref, out_ref,
             buf, obs_v, acoef, bcoef, posv, negv):
    wid = lax.axis_index("s") * 2 + lax.axis_index("c")
    pltpu.sync_copy(pos_ref, posv)
    pltpu.sync_copy(neg_ref, negv)
    pos_vals = [posv[pl.ds(16 * j, 16)] for j in range(8)]
    neg_vals = [negv[pl.ds(16 * j, 16)] for j in range(8)]

    def chunk_body(c, carry):
        k = wid * _CPW + c
        pltpu.sync_copy(node_ref.at[pl.ds(k * _CB, _CB)], buf)
        pltpu.sync_copy(obs_ref.at[k], obs_v)
        for g in range(_OC // 16):
            o = obs_v[pl.ds(16 * g, 16)]
            acoef[pl.ds(16 * g, 16)] = (o == 1).astype(jnp.float32)
            bcoef[pl.ds(16 * g, 16)] = (o == -1).astype(jnp.float32)

        def row_body(r, carry2):
            idxv = jnp.full((16,), r, dtype=jnp.int32)
            av = plsc.load_gather(acoef, [idxv])
            bv = plsc.load_gather(bcoef, [idxv])
            base = r * _D
            for j in range(8):
                sl = pl.ds(base + 16 * j, 16)
                buf[sl] = buf[sl] + av * pos_vals[j] + bv * neg_vals[j]
            return carry2

        lax.fori_loop(0, _RC, row_body, 0)
        pltpu.sync_copy(buf, out_ref.at[pl.ds(k * _CB, _CB)])
        return carry

    lax.fori_loop(0, _CPW, chunk_body, 0)


def _sc_kernel(node_data, pos_test_spike, neg_test_spike, observations):
    nd1 = node_data.reshape(_R * _D)
    obs = observations.reshape(_NCHUNK, _RC).astype(jnp.int32)
    obs_pad = jnp.pad(obs, ((0, 0), (0, _OC - _RC)))

    call = pl.kernel(
        _sc_body,
        out_type=jax.ShapeDtypeStruct((_R * _D,), jnp.float32),
        mesh=plsc.VectorSubcoreMesh(core_axis_name="c", subcore_axis_name="s"),
        compiler_params=pltpu.CompilerParams(needs_layout_passes=False),
        scratch_types=[
            pltpu.VMEM((_CB,), jnp.float32),
            pltpu.VMEM((_OC,), jnp.int32),
            pltpu.VMEM((_OC,), jnp.float32),
            pltpu.VMEM((_OC,), jnp.float32),
            pltpu.VMEM((_D,), jnp.float32),
            pltpu.VMEM((_D,), jnp.float32),
        ],
    )
    out = call(nd1, obs_pad, pos_test_spike, neg_test_spike)
    return out.reshape(_T, _N, _D)


def kernel(node_data, edge_weights, pos_test_spike, neg_test_spike, observations):
    out = _sc_kernel(node_data, pos_test_spike, neg_test_spike, observations)
    return out, edge_weights


# speedup vs baseline: 1.7671x; 1.7671x over previous
import jax
import jax.numpy as jnp
from jax import lax
from jax.experimental import pallas as pl
from jax.experimental.pallas import tpu as pltpu
from jax.experimental.pallas import tpu_sc as plsc

_T = 20
_N = 10000
_D = 128
_R = _T * _N

_BR = 25000
_GRID = _R // _BR

_NW = 32
_RC = 250
_NCHUNK = _R // _RC
_CPW = _NCHUNK // _NW
_CB = _RC * _D
_OC = 256


def _tc_body(obs_ref, nd_ref, spikes_ref, out_ref):
    obs = obs_ref[0]
    a = (obs == 1).astype(jnp.float32)
    b = (obs == -1).astype(jnp.float32)
    coef = jnp.concatenate([a, b], axis=0)
    contrib = jax.lax.dot_general(
        coef, spikes_ref[...],
        dimension_numbers=(((0,), (0,)), ((), ())),
        preferred_element_type=jnp.float32,
    )
    out_ref[...] = nd_ref[...] + contrib


def _tc_kernel(node_data, pos_test_spike, neg_test_spike, observations):
    nd = node_data.reshape(_R, _D)
    obs = observations.reshape(_GRID, 1, _BR).astype(jnp.int32)
    spikes = jnp.stack([pos_test_spike, neg_test_spike], axis=0)

    out = pl.pallas_call(
        _tc_body,
        grid=(_GRID,),
        in_specs=[
            pl.BlockSpec((1, 1, _BR), lambda i: (i, 0, 0)),
            pl.BlockSpec((_BR, _D), lambda i: (i, 0)),
            pl.BlockSpec((2, _D), lambda i: (0, 0)),
        ],
        out_specs=pl.BlockSpec((_BR, _D), lambda i: (i, 0)),
        out_shape=jax.ShapeDtypeStruct((_R, _D), jnp.float32),
        compiler_params=pltpu.CompilerParams(
            dimension_semantics=("arbitrary",),
        ),
    )(obs, nd, spikes)
    return out.reshape(_T, _N, _D)


_NBUF = 3


def _sc_body(node_ref, obs_ref, pos_ref, neg_ref, out_ref,
             bufs, obs_vs, acoef, bcoef, posv, negv, in_sems, out_sems):
    wid = lax.axis_index("s") * 2 + lax.axis_index("c")
    pltpu.sync_copy(pos_ref, posv)
    pltpu.sync_copy(neg_ref, negv)
    pos_vals = [posv[pl.ds(16 * j, 16)] for j in range(8)]
    neg_vals = [negv[pl.ds(16 * j, 16)] for j in range(8)]
    base_chunk = wid * _CPW

    def start_in(c):
        b = c % _NBUF
        k = base_chunk + c
        pltpu.async_copy(node_ref.at[pl.ds(k * _CB, _CB)], bufs[b], in_sems[b])
        pltpu.async_copy(obs_ref.at[k], obs_vs[b], in_sems[b])

    def wait_in(c):
        b = c % _NBUF
        k = base_chunk + c
        pltpu.make_async_copy(node_ref.at[pl.ds(k * _CB, _CB)], bufs[b], in_sems[b]).wait()
        pltpu.make_async_copy(obs_ref.at[k], obs_vs[b], in_sems[b]).wait()

    def start_out(c):
        b = c % _NBUF
        k = base_chunk + c
        pltpu.async_copy(bufs[b], out_ref.at[pl.ds(k * _CB, _CB)], out_sems[b])

    def wait_out(c):
        b = c % _NBUF
        k = base_chunk + c
        pltpu.make_async_copy(bufs[b], out_ref.at[pl.ds(k * _CB, _CB)], out_sems[b]).wait()

    def compute(c):
        b = c % _NBUF
        buf, obs_v = bufs[b], obs_vs[b]
        for g in range(_OC // 16):
            o = obs_v[pl.ds(16 * g, 16)]
            acoef[pl.ds(16 * g, 16)] = (o == 1).astype(jnp.float32)
            bcoef[pl.ds(16 * g, 16)] = (o == -1).astype(jnp.float32)

        def row_body(r, carry2):
            idxv = jnp.full((16,), r, dtype=jnp.int32)
            av = plsc.load_gather(acoef, [idxv])
            bv = plsc.load_gather(bcoef, [idxv])
            base = r * _D
            for j in range(8):
                sl = pl.ds(base + 16 * j, 16)
                buf[sl] = buf[sl] + av * pos_vals[j] + bv * neg_vals[j]
            return carry2

        lax.fori_loop(0, _RC, row_body, 0)

    start_in(0)
    start_in(1)
    for c in range(_CPW):
        wait_in(c)
        compute(c)
        start_out(c)
        if c + 2 < _CPW:
            if c >= 1:
                wait_out(c - 1)
            start_in(c + 2)
    wait_out(_CPW - 2)
    wait_out(_CPW - 1)


def _sc_kernel(node_data, pos_test_spike, neg_test_spike, observations):
    nd1 = node_data.reshape(_R * _D)
    obs = observations.reshape(_NCHUNK, _RC).astype(jnp.int32)
    obs_pad = jnp.pad(obs, ((0, 0), (0, _OC - _RC)))

    call = pl.kernel(
        _sc_body,
        out_type=jax.ShapeDtypeStruct((_R * _D,), jnp.float32),
        mesh=plsc.VectorSubcoreMesh(core_axis_name="c", subcore_axis_name="s"),
        compiler_params=pltpu.CompilerParams(needs_layout_passes=False),
        scratch_types=[
            [pltpu.VMEM((_CB,), jnp.float32) for _ in range(_NBUF)],
            [pltpu.VMEM((_OC,), jnp.int32) for _ in range(_NBUF)],
            pltpu.VMEM((_OC,), jnp.float32),
            pltpu.VMEM((_OC,), jnp.float32),
            pltpu.VMEM((_D,), jnp.float32),
            pltpu.VMEM((_D,), jnp.float32),
            [pltpu.SemaphoreType.DMA for _ in range(_NBUF)],
            [pltpu.SemaphoreType.DMA for _ in range(_NBUF)],
        ],
    )
    out = call(nd1, obs_pad, pos_test_spike, neg_test_spike)
    return out.reshape(_T, _N, _D)


def kernel(node_data, edge_weights, pos_test_spike, neg_test_spike, observations):
    out = _sc_kernel(node_data, pos_test_spike, neg_test_spike, observations)
    return out, edge_weights
